# Initial kernel scaffold; baseline (speedup 1.0000x reference)
import jax
import jax.numpy as jnp
from jax.experimental import pallas as pl


def kernel(x, edge_index, W1, b1, W2, b2, W3, b3):
    raise NotImplementedError("write your pallas kernel here")


# speedup vs baseline: 5.7780x; 5.7780x over previous
import functools

import jax
import jax.numpy as jnp
from jax import lax
from jax.experimental import pallas as pl
from jax.experimental.pallas import tpu as pltpu
from jax.experimental.pallas import tpu_sc as plsc

N = 10000
E = 320000
D = 128
NC = 2
NS = 16
EPT = E // NS
CB = 80
NCHUNK = EPT // CB
NP = 10240
HALF = NP // 2
DROWS = HALF + 128
DRPT = DROWS // NS
HRPT = HALF // NS

_MESH = dict(core_axis_name="c", subcore_axis_name="s", num_cores=NC,
             num_subcores=NS)


def _zero_fill(buf, rows, width):
    z = jnp.zeros((16,), jnp.float32)

    def body(r, carry):
        for k in range(width // 16):
            buf[r, pl.ds(k * 16, 16)] = z
        return carry

    lax.fori_loop(0, rows, body, 0)


def _localize(dst_v, dst_loc, lo):
    for k in range(CB // 16):
        v = dst_v[pl.ds(k * 16, 16)] - lo
        ok = (v >= 0) & (v < HALF)
        dst_loc[pl.ds(k * 16, 16)] = jnp.where(ok, v, HALF)


def _deg_body(dst_hbm, part_hbm, dst_v, dst_loc, ones_v, zbuf, acc, sem):
    cid = lax.axis_index("c")
    sid = lax.axis_index("s")
    base = sid * EPT
    lo = cid * HALF

    one = jnp.full((16,), 1.0, jnp.float32)

    def fill_ones(r, carry):
        ones_v[r, pl.ds(0, 16)] = one
        return carry

    lax.fori_loop(0, CB, fill_ones, 0)
    _zero_fill(zbuf, DRPT, 16)
    pltpu.sync_copy(zbuf, acc.at[pl.ds(sid * DRPT, DRPT)])
    plsc.subcore_barrier()

    def chunk(n, carry):
        off = base + n * CB
        pltpu.sync_copy(dst_hbm.at[pl.ds(off, CB)], dst_v)
        _localize(dst_v, dst_loc, lo)
        pltpu.sync_copy(ones_v, acc.at[dst_loc], add=True)
        return carry

    lax.fori_loop(0, NCHUNK, chunk, 0)
    plsc.subcore_barrier()
    pltpu.sync_copy(acc.at[pl.ds(sid * HRPT, HRPT)],
                    part_hbm.at[pl.ds(cid * HALF + sid * HRPT, HRPT)])


_deg_call = functools.partial(
    pl.kernel,
    out_type=jax.ShapeDtypeStruct((NP, 16), jnp.float32),
    mesh=plsc.VectorSubcoreMesh(**_MESH),
    scratch_types=[
        pltpu.VMEM((CB,), jnp.int32),
        pltpu.VMEM((CB,), jnp.int32),
        pltpu.VMEM((CB, 16), jnp.float32),
        pltpu.VMEM((DRPT, 16), jnp.float32),
        pltpu.VMEM_SHARED((DROWS, 16), jnp.float32),
        pltpu.SemaphoreType.DMA,
    ],
)(_deg_body)


def _agg_body(g_hbm, src_hbm, dst_hbm, part_hbm, src_v, dst_v, dst_loc,
              rows_v, zbuf, acc, sem):
    cid = lax.axis_index("c")
    sid = lax.axis_index("s")
    base = sid * EPT
    lo = cid * HALF

    _zero_fill(zbuf, DRPT, D)
    pltpu.sync_copy(zbuf, acc.at[pl.ds(sid * DRPT, DRPT)])
    plsc.subcore_barrier()

    def chunk(n, carry):
        off = base + n * CB
        pltpu.sync_copy(src_hbm.at[pl.ds(off, CB)], src_v)
        pltpu.sync_copy(dst_hbm.at[pl.ds(off, CB)], dst_v)
        _localize(dst_v, dst_loc, lo)
        pltpu.async_copy(g_hbm.at[src_v], rows_v, sem).wait()
        pltpu.sync_copy(rows_v, acc.at[dst_loc], add=True)
        return carry

    lax.fori_loop(0, NCHUNK, chunk, 0)
    plsc.subcore_barrier()
    pltpu.sync_copy(acc.at[pl.ds(sid * HRPT, HRPT)],
                    part_hbm.at[pl.ds(cid * HALF + sid * HRPT, HRPT)])


_agg_call = functools.partial(
    pl.kernel,
    out_type=jax.ShapeDtypeStruct((NP, D), jnp.float32),
    mesh=plsc.VectorSubcoreMesh(**_MESH),
    scratch_types=[
        pltpu.VMEM((CB,), jnp.int32),
        pltpu.VMEM((CB,), jnp.int32),
        pltpu.VMEM((CB,), jnp.int32),
        pltpu.VMEM((CB, D), jnp.float32),
        pltpu.VMEM((DRPT, D), jnp.float32),
        pltpu.VMEM_SHARED((DROWS, D), jnp.float32),
        pltpu.SemaphoreType.DMA,
    ],
)(_agg_body)


BS = 1000
GRID = N // BS

_DOT = dict(preferred_element_type=jnp.float32,
            precision=jax.lax.Precision.HIGHEST)

_ROW = pl.BlockSpec((BS, D), lambda i: (i, 0))
_FULL = pl.BlockSpec((D, D), lambda i: (0, 0))
_BIAS = pl.BlockSpec((1, D), lambda i: (0, 0))
_OUT = jax.ShapeDtypeStruct((N, D), jnp.float32)


def _dinv_body(degp_ref, dinv_ref):
    deg = jnp.sum(degp_ref[...], axis=1) / 16.0 + 1.0
    dinv_ref[...] = jnp.broadcast_to(lax.rsqrt(deg)[:, None], (BS, D))


def _dinv(degp):
    return pl.pallas_call(
        _dinv_body,
        grid=(GRID,),
        in_specs=[pl.BlockSpec((BS, 16), lambda i: (i, 0))],
        out_specs=_ROW,
        out_shape=_OUT,
    )(degp)


def _mm_body(a_ref, w_ref, dinv_ref, g_ref):
    g_ref[...] = jnp.dot(a_ref[...], w_ref[...], **_DOT) * dinv_ref[...]


def _mm(a, w, dinvb):
    return pl.pallas_call(
        _mm_body,
        grid=(GRID,),
        in_specs=[_ROW, _FULL, _ROW],
        out_specs=_ROW,
        out_shape=_OUT,
    )(a, w, dinvb)


def _combine_body(p_ref, g_ref, dinv_ref, b_ref, a_ref):
    a_ref[...] = jnp.maximum(
        dinv_ref[...] * (p_ref[...] + g_ref[...]) + b_ref[...], 0.0)


def _combine(p, g, dinvb, b):
    return pl.pallas_call(
        _combine_body,
        grid=(GRID,),
        in_specs=[_ROW, _ROW, _ROW, _BIAS],
        out_specs=_ROW,
        out_shape=_OUT,
    )(p, g, dinvb, b)


def kernel(x, edge_index, W1, b1, W2, b2, W3, b3):
    src = edge_index[0].astype(jnp.int32)
    dst = edge_index[1].astype(jnp.int32)

    degp = _deg_call(dst)
    dinvb = _dinv(degp)

    a = x
    for (w, b) in ((W1, b1), (W2, b2), (W3, b3)):
        g = _mm(a, w, dinvb)
        p = _agg_call(g, src, dst)
        a = _combine(p[:N], g, dinvb, b.reshape(1, D))
    return a


# speedup vs baseline: 10.0752x; 1.7437x over previous
import functools

import jax
import jax.numpy as jnp
from jax import lax
from jax.experimental import pallas as pl
from jax.experimental.pallas import tpu as pltpu
from jax.experimental.pallas import tpu_sc as plsc

N = 10000
E = 320000
D = 128
NC = 2
NS = 16
EPT = E // NS
CB = 80
NCHUNK = EPT // CB
NP = 10240
HALF = NP // 2
DROWS = HALF + 128
DRPT = DROWS // NS
HRPT = HALF // NS

_MESH = dict(core_axis_name="c", subcore_axis_name="s", num_cores=NC,
             num_subcores=NS)


def _zero_fill(buf, rows, width):
    z = jnp.zeros((16,), jnp.float32)

    def body(r, carry):
        for k in range(width // 16):
            buf[r, pl.ds(k * 16, 16)] = z
        return carry

    lax.fori_loop(0, rows, body, 0)


def _localize(dst_v, dst_loc, lo):
    for k in range(CB // 16):
        v = dst_v[pl.ds(k * 16, 16)] - lo
        ok = (v >= 0) & (v < HALF)
        dst_loc[pl.ds(k * 16, 16)] = jnp.where(ok, v, HALF)


def _deg_body(dst_hbm, part_hbm, dst_v, dst_loc, ones_v, zbuf, acc, sem):
    cid = lax.axis_index("c")
    sid = lax.axis_index("s")
    base = sid * EPT
    lo = cid * HALF

    one = jnp.full((16,), 1.0, jnp.float32)

    def fill_ones(r, carry):
        ones_v[r, pl.ds(0, 16)] = one
        return carry

    lax.fori_loop(0, CB, fill_ones, 0)
    _zero_fill(zbuf, DRPT, 16)
    pltpu.sync_copy(zbuf, acc.at[pl.ds(sid * DRPT, DRPT)])
    plsc.subcore_barrier()

    def chunk(n, carry):
        off = base + n * CB
        pltpu.sync_copy(dst_hbm.at[pl.ds(off, CB)], dst_v)
        _localize(dst_v, dst_loc, lo)
        pltpu.sync_copy(ones_v, acc.at[dst_loc], add=True)
        return carry

    lax.fori_loop(0, NCHUNK, chunk, 0)
    plsc.subcore_barrier()
    pltpu.sync_copy(acc.at[pl.ds(sid * HRPT, HRPT)],
                    part_hbm.at[pl.ds(cid * HALF + sid * HRPT, HRPT)])


_deg_call = functools.partial(
    pl.kernel,
    out_type=jax.ShapeDtypeStruct((NP, 16), jnp.float32),
    mesh=plsc.VectorSubcoreMesh(**_MESH),
    scratch_types=[
        pltpu.VMEM((CB,), jnp.int32),
        pltpu.VMEM((CB,), jnp.int32),
        pltpu.VMEM((CB, 16), jnp.float32),
        pltpu.VMEM((DRPT, 16), jnp.float32),
        pltpu.VMEM_SHARED((DROWS, 16), jnp.float32),
        pltpu.SemaphoreType.DMA,
    ],
)(_deg_body)


NBUF = 4
OUTER = NCHUNK // NBUF
LEFT = NCHUNK - OUTER * NBUF


def _agg_body(g_hbm, src_hbm, dst_hbm, part_hbm, src_all, dst_all,
              src_loc0, src_loc1, src_loc2, src_loc3,
              dst_loc0, dst_loc1, dst_loc2, dst_loc3,
              rows0, rows1, rows2, rows3,
              acc, sem0, sem1, sem2, sem3):
    cid = lax.axis_index("c")
    sid = lax.axis_index("s")
    base = sid * EPT
    lo = cid * HALF
    src_loc = (src_loc0, src_loc1, src_loc2, src_loc3)
    dst_loc = (dst_loc0, dst_loc1, dst_loc2, dst_loc3)
    rows = (rows0, rows1, rows2, rows3)
    sem = (sem0, sem1, sem2, sem3)

    def stage(n, b):
        for k in range(CB // 16):
            src_loc[b][pl.ds(k * 16, 16)] = src_all[pl.ds(n * CB + k * 16, 16)]
            v = dst_all[pl.ds(n * CB + k * 16, 16)] - lo
            ok = (v >= 0) & (v < HALF)
            dst_loc[b][pl.ds(k * 16, 16)] = jnp.where(ok, v, HALF)

    def fire(n, b):
        stage(n, b)
        return pltpu.async_copy(g_hbm.at[src_loc[b]], rows[b], sem[b])

    def drain(d, b):
        d.wait()
        pltpu.sync_copy(rows[b], acc.at[dst_loc[b]], add=True)

    pltpu.sync_copy(src_hbm.at[pl.ds(base, EPT)], src_all)
    pltpu.sync_copy(dst_hbm.at[pl.ds(base, EPT)], dst_all)

    for b in range(NBUF):
        _zero_fill(rows[b], CB, D)
        pltpu.sync_copy(rows[b], acc.at[pl.ds(sid * DRPT + b * CB, CB)])
    rem = DRPT - NBUF * CB
    if rem:
        pltpu.sync_copy(rows0.at[pl.ds(0, rem)],
                        acc.at[pl.ds(sid * DRPT + NBUF * CB, rem)])

    plsc.subcore_barrier()

    def outer(h, carry):
        ds = [fire(h * NBUF + b, b) for b in range(NBUF)]
        for b in range(NBUF):
            drain(ds[b], b)
        return carry

    lax.fori_loop(0, OUTER, outer, 0)
    if LEFT:
        ds = [fire(OUTER * NBUF + b, b) for b in range(LEFT)]
        for b in range(LEFT):
            drain(ds[b], b)

    plsc.subcore_barrier()
    pltpu.sync_copy(acc.at[pl.ds(sid * HRPT, HRPT)],
                    part_hbm.at[pl.ds(cid * HALF + sid * HRPT, HRPT)])


_agg_call = functools.partial(
    pl.kernel,
    out_type=jax.ShapeDtypeStruct((NP, D), jnp.float32),
    mesh=plsc.VectorSubcoreMesh(**_MESH),
    scratch_types=(
        [pltpu.VMEM((EPT,), jnp.int32)] * 2
        + [pltpu.VMEM((CB,), jnp.int32)] * (2 * NBUF)
        + [pltpu.VMEM((CB, D), jnp.float32)] * NBUF
        + [pltpu.VMEM_SHARED((DROWS, D), jnp.float32)]
        + [pltpu.SemaphoreType.DMA] * NBUF
    ),
)(_agg_body)


BS = 1000
GRID = N // BS

_DOT = dict(preferred_element_type=jnp.float32,
            precision=jax.lax.Precision.HIGHEST)

_ROW = pl.BlockSpec((BS, D), lambda i: (i, 0))
_FULL = pl.BlockSpec((D, D), lambda i: (0, 0))
_BIAS = pl.BlockSpec((1, D), lambda i: (0, 0))
_OUT = jax.ShapeDtypeStruct((N, D), jnp.float32)


def _dinv_body(degp_ref, dinv_ref):
    deg = jnp.sum(degp_ref[...], axis=1) / 16.0 + 1.0
    dinv_ref[...] = jnp.broadcast_to(lax.rsqrt(deg)[:, None], (BS, D))


def _dinv(degp):
    return pl.pallas_call(
        _dinv_body,
        grid=(GRID,),
        in_specs=[pl.BlockSpec((BS, 16), lambda i: (i, 0))],
        out_specs=_ROW,
        out_shape=_OUT,
    )(degp)


def _mm_body(a_ref, w_ref, dinv_ref, g_ref):
    g_ref[...] = jnp.dot(a_ref[...], w_ref[...], **_DOT) * dinv_ref[...]


def _mm(a, w, dinvb):
    return pl.pallas_call(
        _mm_body,
        grid=(GRID,),
        in_specs=[_ROW, _FULL, _ROW],
        out_specs=_ROW,
        out_shape=_OUT,
    )(a, w, dinvb)


def _combine_body(p_ref, g_ref, dinv_ref, b_ref, a_ref):
    a_ref[...] = jnp.maximum(
        dinv_ref[...] * (p_ref[...] + g_ref[...]) + b_ref[...], 0.0)


def _combine(p, g, dinvb, b):
    return pl.pallas_call(
        _combine_body,
        grid=(GRID,),
        in_specs=[_ROW, _ROW, _ROW, _BIAS],
        out_specs=_ROW,
        out_shape=_OUT,
    )(p, g, dinvb, b)


def kernel(x, edge_index, W1, b1, W2, b2, W3, b3):
    src = edge_index[0].astype(jnp.int32)
    dst = edge_index[1].astype(jnp.int32)

    degp = _deg_call(dst)
    dinvb = _dinv(degp)

    a = x
    for (w, b) in ((W1, b1), (W2, b2), (W3, b3)):
        g = _mm(a, w, dinvb)
        p = _agg_call(g, src, dst)
        a = _combine(p[:N], g, dinvb, b.reshape(1, D))
    return a


# speedup vs baseline: 10.3375x; 1.0260x over previous
import functools

import jax
import jax.numpy as jnp
from jax import lax
from jax.experimental import pallas as pl
from jax.experimental.pallas import tpu as pltpu
from jax.experimental.pallas import tpu_sc as plsc

N = 10000
E = 320000
D = 128
NC = 2
NS = 16
EPT = E // NS
CB = 80
NCHUNK = EPT // CB
NP = 10240
HALF = NP // 2
DROWS = HALF + 128
DRPT = DROWS // NS
HRPT = HALF // NS

NBUF = 4
ROUNDS = NCHUNK // NBUF
LEFT = NCHUNK - ROUNDS * NBUF

_MESH = dict(core_axis_name="c", subcore_axis_name="s", num_cores=NC,
             num_subcores=NS)


def _zero_fill(buf, rows, width):
    z = jnp.zeros((16,), jnp.float32)

    def body(r, carry):
        for k in range(width // 16):
            buf[r, pl.ds(k * 16, 16)] = z
        return carry

    lax.fori_loop(0, rows, body, 0)


def _deg_body(dst_hbm, part_hbm, dst_all, dst_loc, ones_v, zbuf, acc, sem):
    cid = lax.axis_index("c")
    sid = lax.axis_index("s")
    base = sid * EPT
    lo = cid * HALF

    pltpu.sync_copy(dst_hbm.at[pl.ds(base, EPT)], dst_all)

    one = jnp.full((16,), 1.0, jnp.float32)

    def fill_ones(r, carry):
        ones_v[r, pl.ds(0, 16)] = one
        return carry

    lax.fori_loop(0, CB, fill_ones, 0)
    _zero_fill(zbuf, DRPT, 16)
    pltpu.sync_copy(zbuf, acc.at[pl.ds(sid * DRPT, DRPT)])
    plsc.subcore_barrier()

    def chunk(n, carry):
        for k in range(CB // 16):
            v = dst_all[pl.ds(n * CB + k * 16, 16)] - lo
            ok = (v >= 0) & (v < HALF)
            dst_loc[pl.ds(k * 16, 16)] = jnp.where(ok, v, HALF)
        pltpu.sync_copy(ones_v, acc.at[dst_loc], add=True)
        return carry

    lax.fori_loop(0, NCHUNK, chunk, 0)
    plsc.subcore_barrier()
    pltpu.sync_copy(acc.at[pl.ds(sid * HRPT, HRPT)],
                    part_hbm.at[pl.ds(cid * HALF + sid * HRPT, HRPT)])


_deg_call = functools.partial(
    pl.kernel,
    out_type=jax.ShapeDtypeStruct((NP, 16), jnp.float32),
    mesh=plsc.VectorSubcoreMesh(**_MESH),
    scratch_types=[
        pltpu.VMEM((EPT,), jnp.int32),
        pltpu.VMEM((CB,), jnp.int32),
        pltpu.VMEM((CB, 16), jnp.float32),
        pltpu.VMEM((DRPT, 16), jnp.float32),
        pltpu.VMEM_SHARED((DROWS, 16), jnp.float32),
        pltpu.SemaphoreType.DMA,
    ],
)(_deg_body)


def _agg_body(g_hbm, src_hbm, dst_hbm, part_hbm, src_all, dst_all,
              src_loc0, src_loc1, src_loc2, src_loc3,
              dst_loc0, dst_loc1, dst_loc2, dst_loc3,
              rows0, rows1, rows2, rows3,
              acc, sem0, sem1, sem2, sem3,
              ssem0, ssem1, ssem2, ssem3):
    cid = lax.axis_index("c")
    sid = lax.axis_index("s")
    base = sid * EPT
    lo = cid * HALF
    src_loc = (src_loc0, src_loc1, src_loc2, src_loc3)
    dst_loc = (dst_loc0, dst_loc1, dst_loc2, dst_loc3)
    rows = (rows0, rows1, rows2, rows3)
    sem = (sem0, sem1, sem2, sem3)
    ssem = (ssem0, ssem1, ssem2, ssem3)

    def stage(n, b):
        for k in range(CB // 16):
            src_loc[b][pl.ds(k * 16, 16)] = src_all[pl.ds(n * CB + k * 16, 16)]
            v = dst_all[pl.ds(n * CB + k * 16, 16)] - lo
            ok = (v >= 0) & (v < HALF)
            dst_loc[b][pl.ds(k * 16, 16)] = jnp.where(ok, v, HALF)

    def fire(n, b):
        stage(n, b)
        return pltpu.async_copy(g_hbm.at[src_loc[b]], rows[b], sem[b])


    pltpu.sync_copy(src_hbm.at[pl.ds(base, EPT)], src_all)
    pltpu.sync_copy(dst_hbm.at[pl.ds(base, EPT)], dst_all)

    nz = DRPT // CB
    for b in range(nz):
        _zero_fill(rows[b], CB, D)
        pltpu.sync_copy(rows[b], acc.at[pl.ds(sid * DRPT + b * CB, CB)])
    rem = DRPT - nz * CB
    if rem:
        pltpu.sync_copy(rows0.at[pl.ds(0, rem)],
                        acc.at[pl.ds(sid * DRPT + nz * CB, rem)])

    plsc.subcore_barrier()

    def round_(n0, nb):
        ds = [fire(n0 + b, b) for b in range(nb)]
        ss = []
        for b in range(nb):
            ds[b].wait()
            ss.append(pltpu.async_copy(rows[b], acc.at[dst_loc[b]], ssem[b],
                                       add=True))
        for s in ss:
            s.wait()

    def outer(h, carry):
        round_(h * NBUF, NBUF)
        return carry

    lax.fori_loop(0, ROUNDS, outer, 0)
    if LEFT:
        round_(ROUNDS * NBUF, LEFT)

    plsc.subcore_barrier()
    pltpu.sync_copy(acc.at[pl.ds(sid * HRPT, HRPT)],
                    part_hbm.at[pl.ds(cid * HALF + sid * HRPT, HRPT)])


_agg_call = functools.partial(
    pl.kernel,
    out_type=jax.ShapeDtypeStruct((NP, D), jnp.float32),
    mesh=plsc.VectorSubcoreMesh(**_MESH),
    scratch_types=(
        [pltpu.VMEM((EPT,), jnp.int32)] * 2
        + [pltpu.VMEM((CB,), jnp.int32)] * (2 * NBUF)
        + [pltpu.VMEM((CB, D), jnp.float32)] * NBUF
        + [pltpu.VMEM_SHARED((DROWS, D), jnp.float32)]
        + [pltpu.SemaphoreType.DMA] * (2 * NBUF)
    ),
)(_agg_body)


BS = 1000
GRID = N // BS

_DOT = dict(preferred_element_type=jnp.float32,
            precision=jax.lax.Precision.HIGHEST)

_ROW = pl.BlockSpec((BS, D), lambda i: (i, 0))
_FULL = pl.BlockSpec((D, D), lambda i: (0, 0))
_BIAS = pl.BlockSpec((1, D), lambda i: (0, 0))
_OUT = jax.ShapeDtypeStruct((N, D), jnp.float32)


def _dinv_body(degp_ref, dinv_ref):
    deg = jnp.sum(degp_ref[...], axis=1) / 16.0 + 1.0
    dinv_ref[...] = jnp.broadcast_to(lax.rsqrt(deg)[:, None], (BS, D))


def _dinv(degp):
    return pl.pallas_call(
        _dinv_body,
        grid=(GRID,),
        in_specs=[pl.BlockSpec((BS, 16), lambda i: (i, 0))],
        out_specs=_ROW,
        out_shape=_OUT,
    )(degp)


def _mm_body(a_ref, w_ref, dinv_ref, g_ref):
    g_ref[...] = jnp.dot(a_ref[...], w_ref[...], **_DOT) * dinv_ref[...]


def _mm(a, w, dinvb):
    return pl.pallas_call(
        _mm_body,
        grid=(GRID,),
        in_specs=[_ROW, _FULL, _ROW],
        out_specs=_ROW,
        out_shape=_OUT,
    )(a, w, dinvb)


def _combine_body(p_ref, g_ref, dinv_ref, b_ref, a_ref):
    a_ref[...] = jnp.maximum(
        dinv_ref[...] * (p_ref[...] + g_ref[...]) + b_ref[...], 0.0)


def _combine(p, g, dinvb, b):
    return pl.pallas_call(
        _combine_body,
        grid=(GRID,),
        in_specs=[_ROW, _ROW, _ROW, _BIAS],
        out_specs=_ROW,
        out_shape=_OUT,
    )(p, g, dinvb, b)


def kernel(x, edge_index, W1, b1, W2, b2, W3, b3):
    src = edge_index[0].astype(jnp.int32)
    dst = edge_index[1].astype(jnp.int32)

    degp = _deg_call(dst)
    dinvb = _dinv(degp)

    a = x
    for (w, b) in ((W1, b1), (W2, b2), (W3, b3)):
        g = _mm(a, w, dinvb)
        p = _agg_call(g, src, dst)
        a = _combine(p[:N], g, dinvb, b.reshape(1, D))
    return a


# speedup vs baseline: 10.7767x; 1.0425x over previous
import functools

import jax
import jax.numpy as jnp
from jax import lax
from jax.experimental import pallas as pl
from jax.experimental.pallas import tpu as pltpu
from jax.experimental.pallas import tpu_sc as plsc

N = 10000
E = 320000
D = 128
NC = 2
NS = 16
EPT = E // NS
CB = 80
NCHUNK = EPT // CB
NP = 10240
HALF = NP // 2
DROWS = HALF + 128
DRPT = DROWS // NS
HRPT = HALF // NS

ACB = 32
ANCHUNK = EPT // ACB
NBUF = 10
ROUNDS = ANCHUNK // NBUF
LEFT = ANCHUNK - ROUNDS * NBUF

_MESH = dict(core_axis_name="c", subcore_axis_name="s", num_cores=NC,
             num_subcores=NS)


def _zero_fill(buf, rows, width):
    z = jnp.zeros((16,), jnp.float32)

    def body(r, carry):
        for k in range(width // 16):
            buf[r, pl.ds(k * 16, 16)] = z
        return carry

    lax.fori_loop(0, rows, body, 0)


DNB = 8
DROUNDS = NCHUNK // DNB
DLEFT = NCHUNK - DROUNDS * DNB


def _deg_body(dst_hbm, part_hbm, *scr):
    dst_all = scr[0]
    dst_loc = scr[1:1 + DNB]
    ones_v = scr[1 + DNB]
    zbuf = scr[2 + DNB]
    acc = scr[3 + DNB]
    sem = scr[4 + DNB:4 + 2 * DNB]
    cid = lax.axis_index("c")
    sid = lax.axis_index("s")
    base = sid * EPT
    lo = cid * HALF

    pltpu.sync_copy(dst_hbm.at[pl.ds(base, EPT)], dst_all)

    one = jnp.full((16,), 1.0, jnp.float32)

    def fill_ones(r, carry):
        ones_v[r, pl.ds(0, 16)] = one
        return carry

    lax.fori_loop(0, CB, fill_ones, 0)
    _zero_fill(zbuf, DRPT, 16)
    pltpu.sync_copy(zbuf, acc.at[pl.ds(sid * DRPT, DRPT)])
    plsc.subcore_barrier()

    def round_(n0, nb):
        ss = []
        for b in range(nb):
            n = n0 + b
            for k in range(CB // 16):
                v = dst_all[pl.ds(n * CB + k * 16, 16)] - lo
                ok = (v >= 0) & (v < HALF)
                dst_loc[b][pl.ds(k * 16, 16)] = jnp.where(ok, v, HALF)
            ss.append(pltpu.async_copy(ones_v, acc.at[dst_loc[b]], sem[b],
                                       add=True))
        for s in ss:
            s.wait()

    def outer(h, carry):
        round_(h * DNB, DNB)
        return carry

    lax.fori_loop(0, DROUNDS, outer, 0)
    if DLEFT:
        round_(DROUNDS * DNB, DLEFT)
    plsc.subcore_barrier()
    pltpu.sync_copy(acc.at[pl.ds(sid * HRPT, HRPT)],
                    part_hbm.at[pl.ds(cid * HALF + sid * HRPT, HRPT)])


_deg_call = functools.partial(
    pl.kernel,
    out_type=jax.ShapeDtypeStruct((NP, 16), jnp.float32),
    mesh=plsc.VectorSubcoreMesh(**_MESH),
    scratch_types=(
        [pltpu.VMEM((EPT,), jnp.int32)]
        + [pltpu.VMEM((CB,), jnp.int32)] * DNB
        + [pltpu.VMEM((CB, 16), jnp.float32)]
        + [pltpu.VMEM((DRPT, 16), jnp.float32)]
        + [pltpu.VMEM_SHARED((DROWS, 16), jnp.float32)]
        + [pltpu.SemaphoreType.DMA] * DNB
    ),
)(_deg_body)


def _agg_body(g_hbm, src_hbm, dst_hbm, nr_hbm, part_hbm, *scr):
    src_all, dst_all = scr[0], scr[1]
    nr_v = scr[2]
    src_loc = scr[3:3 + NBUF]
    dst_loc = scr[3 + NBUF:3 + 2 * NBUF]
    rows = scr[3 + 2 * NBUF:3 + 3 * NBUF]
    sem = scr[3 + 3 * NBUF:3 + 4 * NBUF]
    ssem = scr[3 + 4 * NBUF:3 + 5 * NBUF]
    cid = lax.axis_index("c")
    sid = lax.axis_index("s")
    base = sid * EPT
    lo = cid * HALF

    def stage(n, b):
        for k in range(ACB // 16):
            src_loc[b][pl.ds(k * 16, 16)] = src_all[pl.ds(n * ACB + k * 16, 16)]
            v = dst_all[pl.ds(n * ACB + k * 16, 16)] - lo
            ok = (v >= 0) & (v < HALF)
            dst_loc[b][pl.ds(k * 16, 16)] = jnp.where(ok, v, HALF)

    def fire(n, b):
        stage(n, b)
        return pltpu.async_copy(g_hbm.at[src_loc[b]], rows[b], sem[b])

    pltpu.sync_copy(src_hbm.at[pl.ds(base, EPT)], src_all)
    pltpu.sync_copy(dst_hbm.at[pl.ds(base, EPT)], dst_all)
    pltpu.sync_copy(nr_hbm.at[pl.ds(0, 16)], nr_v)

    nz = DRPT // ACB
    for b in range(nz):
        _zero_fill(rows[b], ACB, D)
        pltpu.sync_copy(rows[b], acc_of(scr).at[pl.ds(sid * DRPT + b * ACB, ACB)])
    rem = DRPT - nz * ACB
    if rem:
        pltpu.sync_copy(rows[0].at[pl.ds(0, rem)],
                        acc_of(scr).at[pl.ds(sid * DRPT + nz * ACB, rem)])

    plsc.subcore_barrier()
    acc = acc_of(scr)

    def round_(n0, nb):
        ds = [fire(n0 + b, b) for b in range(nb)]
        ss = []
        for b in range(nb):
            ds[b].wait()
            ss.append(pltpu.async_copy(rows[b], acc.at[dst_loc[b]], ssem[b],
                                       add=True))
        for s in ss:
            s.wait()

    def outer(h, carry):
        round_(h * NBUF, NBUF)
        return carry

    nr = nr_v[pl.ds(0, 16)][0]
    lax.fori_loop(0, nr, outer, 0)
    if LEFT:
        round_(ROUNDS * NBUF, LEFT)

    plsc.subcore_barrier()
    pltpu.sync_copy(acc.at[pl.ds(sid * HRPT, HRPT)],
                    part_hbm.at[pl.ds(cid * HALF + sid * HRPT, HRPT)])


def acc_of(scr):
    return scr[3 + 5 * NBUF]


_agg_call = functools.partial(
    pl.kernel,
    out_type=jax.ShapeDtypeStruct((NP, D), jnp.float32),
    mesh=plsc.VectorSubcoreMesh(**_MESH),
    scratch_types=(
        [pltpu.VMEM((EPT,), jnp.int32)] * 2
        + [pltpu.VMEM((16,), jnp.int32)]
        + [pltpu.VMEM((ACB,), jnp.int32)] * (2 * NBUF)
        + [pltpu.VMEM((ACB, D), jnp.float32)] * NBUF
        + [pltpu.SemaphoreType.DMA] * (2 * NBUF)
        + [pltpu.VMEM_SHARED((DROWS, D), jnp.float32)]
    ),
)(_agg_body)


BS = 1000
GRID = N // BS

_DOT = dict(preferred_element_type=jnp.float32,
            precision=jax.lax.Precision.HIGHEST)

_ROW = pl.BlockSpec((BS, D), lambda i: (i, 0))
_FULL = pl.BlockSpec((D, D), lambda i: (0, 0))
_BIAS = pl.BlockSpec((1, D), lambda i: (0, 0))
_OUT = jax.ShapeDtypeStruct((N, D), jnp.float32)


def _dinv_body(degp_ref, dinv_ref):
    deg = jnp.sum(degp_ref[...], axis=1) / 16.0 + 1.0
    dinv_ref[...] = jnp.broadcast_to(lax.rsqrt(deg)[:, None], (BS, D))


def _dinv(degp):
    return pl.pallas_call(
        _dinv_body,
        grid=(GRID,),
        in_specs=[pl.BlockSpec((BS, 16), lambda i: (i, 0))],
        out_specs=_ROW,
        out_shape=_OUT,
    )(degp)


def _mm_body(a_ref, w_ref, dinv_ref, g_ref):
    g_ref[...] = jnp.dot(a_ref[...], w_ref[...], **_DOT) * dinv_ref[...]


def _mm(a, w, dinvb):
    return pl.pallas_call(
        _mm_body,
        grid=(GRID,),
        in_specs=[_ROW, _FULL, _ROW],
        out_specs=_ROW,
        out_shape=_OUT,
    )(a, w, dinvb)


def _combine_body(p_ref, g_ref, dinv_ref, b_ref, a_ref):
    a_ref[...] = jnp.maximum(
        dinv_ref[...] * (p_ref[...] + g_ref[...]) + b_ref[...], 0.0)


def _combine(p, g, dinvb, b):
    return pl.pallas_call(
        _combine_body,
        grid=(GRID,),
        in_specs=[_ROW, _ROW, _ROW, _BIAS],
        out_specs=_ROW,
        out_shape=_OUT,
    )(p, g, dinvb, b)


def kernel(x, edge_index, W1, b1, W2, b2, W3, b3):
    src = edge_index[0].astype(jnp.int32)
    dst = edge_index[1].astype(jnp.int32)

    degp = _deg_call(dst)
    nrarr = jnp.full((16,), ROUNDS, jnp.int32)
    dinvb = _dinv(degp)

    a = x
    for (w, b) in ((W1, b1), (W2, b2), (W3, b3)):
        g = _mm(a, w, dinvb)
        p = _agg_call(g, src, dst, nrarr)
        a = _combine(p[:N], g, dinvb, b.reshape(1, D))
    return a


# speedup vs baseline: 11.8412x; 1.0988x over previous
import functools

import jax
import jax.numpy as jnp
from jax import lax
from jax.experimental import pallas as pl
from jax.experimental.pallas import tpu as pltpu
from jax.experimental.pallas import tpu_sc as plsc

N = 10000
E = 320000
D = 128
NC = 2
NS = 16
EPT = E // NS
CB = 80
NCHUNK = EPT // CB
NP = 10240
HALF = NP // 2
DROWS = HALF + 128
DRPT = DROWS // NS
HRPT = HALF // NS

ACB = 32
ANCHUNK = EPT // ACB
NBUF = 10
ROUNDS = ANCHUNK // NBUF
LEFT = ANCHUNK - ROUNDS * NBUF

_MESH = dict(core_axis_name="c", subcore_axis_name="s", num_cores=NC,
             num_subcores=NS)


def _zero_fill(buf, rows, width):
    z = jnp.zeros((16,), jnp.float32)

    def body(r, carry):
        for k in range(width // 16):
            buf[r, pl.ds(k * 16, 16)] = z
        return carry

    lax.fori_loop(0, rows, body, 0)


DNB = 8
DROUNDS = NCHUNK // DNB
DLEFT = NCHUNK - DROUNDS * DNB


def _deg_body(dst_hbm, part_hbm, *scr):
    dst_all = scr[0]
    dst_loc = scr[1:1 + DNB]
    ones_v = scr[1 + DNB]
    zbuf = scr[2 + DNB]
    acc = scr[3 + DNB]
    sem = scr[4 + DNB:4 + 2 * DNB]
    cid = lax.axis_index("c")
    sid = lax.axis_index("s")
    base = sid * EPT
    lo = cid * HALF
    dump = HALF + sid * 8

    pltpu.sync_copy(dst_hbm.at[pl.ds(base, EPT)], dst_all)

    one = jnp.full((16,), 1.0, jnp.float32)

    def fill_ones(r, carry):
        ones_v[r, pl.ds(0, 16)] = one
        return carry

    lax.fori_loop(0, CB, fill_ones, 0)
    _zero_fill(zbuf, DRPT, 16)
    pltpu.sync_copy(zbuf, acc.at[pl.ds(sid * DRPT, DRPT)])
    plsc.subcore_barrier()

    def round_(n0, nb):
        ss = []
        for b in range(nb):
            n = n0 + b
            for k in range(CB // 16):
                v = dst_all[pl.ds(n * CB + k * 16, 16)] - lo
                ok = (v >= 0) & (v < HALF)
                dst_loc[b][pl.ds(k * 16, 16)] = jnp.where(ok, v, dump)
            ss.append(pltpu.async_copy(ones_v, acc.at[dst_loc[b]], sem[b],
                                       add=True))
        for s in ss:
            s.wait()

    def outer(h, carry):
        round_(h * DNB, DNB)
        return carry

    lax.fori_loop(0, DROUNDS, outer, 0)
    if DLEFT:
        round_(DROUNDS * DNB, DLEFT)
    plsc.subcore_barrier()
    pltpu.sync_copy(acc.at[pl.ds(sid * HRPT, HRPT)],
                    part_hbm.at[pl.ds(cid * HALF + sid * HRPT, HRPT)])


_deg_call = functools.partial(
    pl.kernel,
    out_type=jax.ShapeDtypeStruct((NP, 16), jnp.float32),
    mesh=plsc.VectorSubcoreMesh(**_MESH),
    scratch_types=(
        [pltpu.VMEM((EPT,), jnp.int32)]
        + [pltpu.VMEM((CB,), jnp.int32)] * DNB
        + [pltpu.VMEM((CB, 16), jnp.float32)]
        + [pltpu.VMEM((DRPT, 16), jnp.float32)]
        + [pltpu.VMEM_SHARED((DROWS, 16), jnp.float32)]
        + [pltpu.SemaphoreType.DMA] * DNB
    ),
)(_deg_body)


def _agg_body(g_hbm, src_hbm, dst_hbm, nr_hbm, part_hbm, *scr):
    src_all, dst_all = scr[0], scr[1]
    nr_v = scr[2]
    src_loc = scr[3:3 + NBUF]
    dst_loc = scr[3 + NBUF:3 + 2 * NBUF]
    rows = scr[3 + 2 * NBUF:3 + 3 * NBUF]
    sem = scr[3 + 3 * NBUF:3 + 4 * NBUF]
    ssem = scr[3 + 4 * NBUF:3 + 5 * NBUF]
    cid = lax.axis_index("c")
    sid = lax.axis_index("s")
    base = sid * EPT
    lo = cid * HALF
    dump = HALF + sid * 8

    def stage(n, b):
        for k in range(ACB // 16):
            src_loc[b][pl.ds(k * 16, 16)] = src_all[pl.ds(n * ACB + k * 16, 16)]
            v = dst_all[pl.ds(n * ACB + k * 16, 16)] - lo
            ok = (v >= 0) & (v < HALF)
            dst_loc[b][pl.ds(k * 16, 16)] = jnp.where(ok, v, dump)

    def fire(n, b):
        stage(n, b)
        return pltpu.async_copy(g_hbm.at[src_loc[b]], rows[b], sem[b])

    pltpu.sync_copy(src_hbm.at[pl.ds(base, EPT)], src_all)
    pltpu.sync_copy(dst_hbm.at[pl.ds(base, EPT)], dst_all)
    pltpu.sync_copy(nr_hbm.at[pl.ds(0, 16)], nr_v)

    nz = DRPT // ACB
    for b in range(NBUF):
        _zero_fill(rows[b], ACB, D)
    for j in range(nz):
        pltpu.sync_copy(rows[j % NBUF],
                        acc_of(scr).at[pl.ds(sid * DRPT + j * ACB, ACB)])
    rem = DRPT - nz * ACB
    if rem:
        pltpu.sync_copy(rows[0].at[pl.ds(0, rem)],
                        acc_of(scr).at[pl.ds(sid * DRPT + nz * ACB, rem)])

    plsc.subcore_barrier()
    acc = acc_of(scr)

    def round_(n0, nb):
        ds = [fire(n0 + b, b) for b in range(nb)]
        ss = []
        for b in range(nb):
            ds[b].wait()
            ss.append(pltpu.async_copy(rows[b], acc.at[dst_loc[b]], ssem[b],
                                       add=True))
        for s in ss:
            s.wait()

    def outer(h, carry):
        round_(h * NBUF, NBUF)
        return carry

    nr = nr_v[pl.ds(0, 16)][0]
    lax.fori_loop(0, nr, outer, 0)
    if LEFT:
        round_(ROUNDS * NBUF, LEFT)

    plsc.subcore_barrier()
    pltpu.sync_copy(acc.at[pl.ds(sid * HRPT, HRPT)],
                    part_hbm.at[pl.ds(cid * HALF + sid * HRPT, HRPT)])


def acc_of(scr):
    return scr[3 + 5 * NBUF]


_agg_call = functools.partial(
    pl.kernel,
    out_type=jax.ShapeDtypeStruct((NP, D), jnp.float32),
    mesh=plsc.VectorSubcoreMesh(**_MESH),
    scratch_types=(
        [pltpu.VMEM((EPT,), jnp.int32)] * 2
        + [pltpu.VMEM((16,), jnp.int32)]
        + [pltpu.VMEM((ACB,), jnp.int32)] * (2 * NBUF)
        + [pltpu.VMEM((ACB, D), jnp.float32)] * NBUF
        + [pltpu.SemaphoreType.DMA] * (2 * NBUF)
        + [pltpu.VMEM_SHARED((DROWS, D), jnp.float32)]
    ),
)(_agg_body)


BS = 1000
GRID = N // BS

_DOT = dict(preferred_element_type=jnp.float32,
            precision=jax.lax.Precision.HIGHEST)

_ROW = pl.BlockSpec((BS, D), lambda i: (i, 0))
_FULL = pl.BlockSpec((D, D), lambda i: (0, 0))
_BIAS = pl.BlockSpec((1, D), lambda i: (0, 0))
_OUT = jax.ShapeDtypeStruct((N, D), jnp.float32)


def _dinv_body(degp_ref, dinv_ref):
    deg = jnp.sum(degp_ref[...], axis=1) / 16.0 + 1.0
    dinv_ref[...] = jnp.broadcast_to(lax.rsqrt(deg)[:, None], (BS, D))


def _dinv(degp):
    return pl.pallas_call(
        _dinv_body,
        grid=(GRID,),
        in_specs=[pl.BlockSpec((BS, 16), lambda i: (i, 0))],
        out_specs=_ROW,
        out_shape=_OUT,
    )(degp)


def _mm_body(a_ref, w_ref, dinv_ref, g_ref):
    g_ref[...] = jnp.dot(a_ref[...], w_ref[...], **_DOT) * dinv_ref[...]


def _mm(a, w, dinvb):
    return pl.pallas_call(
        _mm_body,
        grid=(GRID,),
        in_specs=[_ROW, _FULL, _ROW],
        out_specs=_ROW,
        out_shape=_OUT,
    )(a, w, dinvb)


def _combine_body(p_ref, g_ref, dinv_ref, b_ref, a_ref):
    a_ref[...] = jnp.maximum(
        dinv_ref[...] * (p_ref[...] + g_ref[...]) + b_ref[...], 0.0)


def _combine(p, g, dinvb, b):
    return pl.pallas_call(
        _combine_body,
        grid=(GRID,),
        in_specs=[_ROW, _ROW, _ROW, _BIAS],
        out_specs=_ROW,
        out_shape=_OUT,
    )(p, g, dinvb, b)


def kernel(x, edge_index, W1, b1, W2, b2, W3, b3):
    src = edge_index[0].astype(jnp.int32)
    dst = edge_index[1].astype(jnp.int32)

    degp = _deg_call(dst)
    nrarr = jnp.full((16,), ROUNDS, jnp.int32)
    dinvb = _dinv(degp)

    a = x
    for (w, b) in ((W1, b1), (W2, b2), (W3, b3)):
        g = _mm(a, w, dinvb)
        p = _agg_call(g, src, dst, nrarr)
        a = _combine(p[:N], g, dinvb, b.reshape(1, D))
    return a


# speedup vs baseline: 12.0814x; 1.0203x over previous
import functools

import jax
import jax.numpy as jnp
from jax import lax
from jax.experimental import pallas as pl
from jax.experimental.pallas import tpu as pltpu
from jax.experimental.pallas import tpu_sc as plsc

N = 10000
E = 320000
D = 128
NC = 2
NS = 16
EPT = E // NS
CB = 80
NCHUNK = EPT // CB
NP = 10240
HALF = NP // 2
DROWS = HALF + 128
DRPT = DROWS // NS
HRPT = HALF // NS

ACB = 32
ANCHUNK = EPT // ACB
NBUF = 10
ROUNDS = ANCHUNK // NBUF
LEFT = ANCHUNK - ROUNDS * NBUF

_MESH = dict(core_axis_name="c", subcore_axis_name="s", num_cores=NC,
             num_subcores=NS)


def _zero_fill(buf, rows, width):
    z = jnp.zeros((16,), jnp.float32)

    def body(r, carry):
        for k in range(width // 16):
            buf[r, pl.ds(k * 16, 16)] = z
        return carry

    lax.fori_loop(0, rows, body, 0)


DNB = 8
DROUNDS = NCHUNK // DNB
DLEFT = NCHUNK - DROUNDS * DNB


def _deg_body(dst_hbm, part_hbm, *scr):
    dst_all = scr[0]
    dst_loc = scr[1:1 + DNB]
    ones_v = scr[1 + DNB]
    zbuf = scr[2 + DNB]
    acc = scr[3 + DNB]
    sem = scr[4 + DNB:4 + 2 * DNB]
    cid = lax.axis_index("c")
    sid = lax.axis_index("s")
    base = sid * EPT
    lo = cid * HALF
    dump = HALF + sid * 8

    pltpu.sync_copy(dst_hbm.at[pl.ds(base, EPT)], dst_all)

    one = jnp.full((16,), 1.0, jnp.float32)

    def fill_ones(r, carry):
        ones_v[r, pl.ds(0, 16)] = one
        return carry

    lax.fori_loop(0, CB, fill_ones, 0)
    _zero_fill(zbuf, DRPT, 16)
    pltpu.sync_copy(zbuf, acc.at[pl.ds(sid * DRPT, DRPT)])
    plsc.subcore_barrier()

    def round_(n0, nb):
        ss = []
        for b in range(nb):
            n = n0 + b
            for k in range(CB // 16):
                v = dst_all[pl.ds(n * CB + k * 16, 16)] - lo
                ok = (v >= 0) & (v < HALF)
                dst_loc[b][pl.ds(k * 16, 16)] = jnp.where(ok, v, dump)
            ss.append(pltpu.async_copy(ones_v, acc.at[dst_loc[b]], sem[b],
                                       add=True))
        for s in ss:
            s.wait()

    def outer(h, carry):
        round_(h * DNB, DNB)
        return carry

    lax.fori_loop(0, DROUNDS, outer, 0)
    if DLEFT:
        round_(DROUNDS * DNB, DLEFT)
    plsc.subcore_barrier()
    pltpu.sync_copy(acc.at[pl.ds(sid * HRPT, HRPT)],
                    part_hbm.at[pl.ds(cid * HALF + sid * HRPT, HRPT)])


_deg_call = functools.partial(
    pl.kernel,
    out_type=jax.ShapeDtypeStruct((NP, 16), jnp.float32),
    mesh=plsc.VectorSubcoreMesh(**_MESH),
    scratch_types=(
        [pltpu.VMEM((EPT,), jnp.int32)]
        + [pltpu.VMEM((CB,), jnp.int32)] * DNB
        + [pltpu.VMEM((CB, 16), jnp.float32)]
        + [pltpu.VMEM((DRPT, 16), jnp.float32)]
        + [pltpu.VMEM_SHARED((DROWS, 16), jnp.float32)]
        + [pltpu.SemaphoreType.DMA] * DNB
    ),
)(_deg_body)


def _agg_body(g_hbm, src_hbm, dst_hbm, nr_hbm, part_hbm, *scr):
    src_all, dst_all = scr[0], scr[1]
    nr_v = scr[2]
    src_loc = scr[3:3 + NBUF]
    dst_loc = scr[3 + NBUF:3 + 2 * NBUF]
    rows = scr[3 + 2 * NBUF:3 + 3 * NBUF]
    sem = scr[3 + 3 * NBUF:3 + 4 * NBUF]
    ssem = scr[3 + 4 * NBUF:3 + 5 * NBUF]
    cid = lax.axis_index("c")
    sid = lax.axis_index("s")
    base = sid * EPT
    lo = cid * HALF
    dump = HALF + sid * 8

    def stage(n, b):
        for k in range(ACB // 16):
            src_loc[b][pl.ds(k * 16, 16)] = src_all[pl.ds(n * ACB + k * 16, 16)]
            v = dst_all[pl.ds(n * ACB + k * 16, 16)] - lo
            ok = (v >= 0) & (v < HALF)
            dst_loc[b][pl.ds(k * 16, 16)] = jnp.where(ok, v, dump)

    def fire(n, b):
        stage(n, b)
        return pltpu.async_copy(g_hbm.at[src_loc[b]], rows[b], sem[b])

    pltpu.sync_copy(src_hbm.at[pl.ds(base, EPT)], src_all)
    pltpu.sync_copy(dst_hbm.at[pl.ds(base, EPT)], dst_all)
    pltpu.sync_copy(nr_hbm.at[pl.ds(0, 16)], nr_v)

    nz = DRPT // ACB
    for b in range(NBUF):
        _zero_fill(rows[b], ACB, D)
    for j in range(nz):
        pltpu.sync_copy(rows[j % NBUF],
                        acc_of(scr).at[pl.ds(sid * DRPT + j * ACB, ACB)])
    rem = DRPT - nz * ACB
    if rem:
        pltpu.sync_copy(rows[0].at[pl.ds(0, rem)],
                        acc_of(scr).at[pl.ds(sid * DRPT + nz * ACB, rem)])

    plsc.subcore_barrier()
    acc = acc_of(scr)

    def round_(n0, nb):
        ds = [fire(n0 + b, b) for b in range(nb)]
        ss = []
        for b in range(nb):
            ds[b].wait()
            ss.append(pltpu.async_copy(rows[b], acc.at[dst_loc[b]], ssem[b],
                                       add=True))
        for s in ss:
            s.wait()

    def outer(h, carry):
        round_(h * NBUF, NBUF)
        return carry

    nr = nr_v[pl.ds(0, 16)][0]
    lax.fori_loop(0, nr, outer, 0)
    if LEFT:
        round_(ROUNDS * NBUF, LEFT)

    plsc.subcore_barrier()
    pltpu.sync_copy(acc.at[pl.ds(sid * HRPT, HRPT)],
                    part_hbm.at[pl.ds(cid * HALF + sid * HRPT, HRPT)])


def acc_of(scr):
    return scr[3 + 5 * NBUF]


_agg_call = functools.partial(
    pl.kernel,
    out_type=jax.ShapeDtypeStruct((NP, D), jnp.float32),
    mesh=plsc.VectorSubcoreMesh(**_MESH),
    scratch_types=(
        [pltpu.VMEM((EPT,), jnp.int32)] * 2
        + [pltpu.VMEM((16,), jnp.int32)]
        + [pltpu.VMEM((ACB,), jnp.int32)] * (2 * NBUF)
        + [pltpu.VMEM((ACB, D), jnp.float32)] * NBUF
        + [pltpu.SemaphoreType.DMA] * (2 * NBUF)
        + [pltpu.VMEM_SHARED((DROWS, D), jnp.float32)]
    ),
)(_agg_body)


BS = 1000
GRID = N // BS

_DOT = dict(preferred_element_type=jnp.float32,
            precision=jax.lax.Precision.HIGHEST)

_ROW = pl.BlockSpec((BS, D), lambda i: (i, 0))
_FULL = pl.BlockSpec((D, D), lambda i: (0, 0))
_BIAS = pl.BlockSpec((1, D), lambda i: (0, 0))
_OUT = jax.ShapeDtypeStruct((N, D), jnp.float32)


def _dinv_body(degp_ref, dinv_ref):
    deg = jnp.sum(degp_ref[...], axis=1) / 16.0 + 1.0
    dinv_ref[...] = jnp.broadcast_to(lax.rsqrt(deg)[:, None], (BS, D))


def _dinv(degp):
    return pl.pallas_call(
        _dinv_body,
        grid=(GRID,),
        in_specs=[pl.BlockSpec((BS, 16), lambda i: (i, 0))],
        out_specs=_ROW,
        out_shape=_OUT,
    )(degp)


def _mm_body(a_ref, w_ref, dinv_ref, g_ref):
    g_ref[...] = jnp.dot(a_ref[...], w_ref[...], **_DOT) * dinv_ref[...]


def _mm(a, w, dinvb):
    return pl.pallas_call(
        _mm_body,
        grid=(GRID,),
        in_specs=[_ROW, _FULL, _ROW],
        out_specs=_ROW,
        out_shape=_OUT,
    )(a, w, dinvb)


def _combine_body(p_ref, g_ref, dinv_ref, b_ref, a_ref):
    a_ref[...] = jnp.maximum(
        dinv_ref[...] * (p_ref[...] + g_ref[...]) + b_ref[...], 0.0)


def _combine(p, g, dinvb, b):
    return pl.pallas_call(
        _combine_body,
        grid=(GRID,),
        in_specs=[_ROW, _ROW, _ROW, _BIAS],
        out_specs=_ROW,
        out_shape=_OUT,
    )(p, g, dinvb, b)


def _layer_body(p_ref, g_ref, dinv_ref, b_ref, w_ref, gn_ref):
    dinv = dinv_ref[...]
    a = jnp.maximum(dinv * (p_ref[...] + g_ref[...]) + b_ref[...], 0.0)
    gn_ref[...] = jnp.dot(a, w_ref[...], **_DOT) * dinv


def _layer(p, g, dinvb, b, w):
    return pl.pallas_call(
        _layer_body,
        grid=(GRID,),
        in_specs=[_ROW, _ROW, _ROW, _BIAS, _FULL],
        out_specs=_ROW,
        out_shape=_OUT,
    )(p, g, dinvb, b, w)


def kernel(x, edge_index, W1, b1, W2, b2, W3, b3):
    src = edge_index[0].astype(jnp.int32)
    dst = edge_index[1].astype(jnp.int32)

    degp = _deg_call(dst)
    nrarr = jnp.full((16,), ROUNDS, jnp.int32)
    dinvb = _dinv(degp)

    g = _mm(x, W1, dinvb)
    p = _agg_call(g, src, dst, nrarr)
    for (b, w) in ((b1, W2), (b2, W3)):
        g = _layer(p[:N], g, dinvb, b.reshape(1, D), w)
        p = _agg_call(g, src, dst, nrarr)
    return _combine(p[:N], g, dinvb, b3.reshape(1, D))


# speedup vs baseline: 12.0974x; 1.0013x over previous
import functools

import jax
import jax.numpy as jnp
from jax import lax
from jax.experimental import pallas as pl
from jax.experimental.pallas import tpu as pltpu
from jax.experimental.pallas import tpu_sc as plsc

N = 10000
E = 320000
D = 128
NC = 2
NS = 16
EPT = E // NS
CB = 80
NCHUNK = EPT // CB
NP = 10240
HALF = NP // 2
DROWS = HALF + 128
DRPT = DROWS // NS
HRPT = HALF // NS

ACB = 32
ANCHUNK = EPT // ACB
NBUF = 11
ROUNDS = ANCHUNK // NBUF
LEFT = ANCHUNK - ROUNDS * NBUF

_MESH = dict(core_axis_name="c", subcore_axis_name="s", num_cores=NC,
             num_subcores=NS)


def _zero_fill(buf, rows, width):
    z = jnp.zeros((16,), jnp.float32)

    def body(r, carry):
        for k in range(width // 16):
            buf[r, pl.ds(k * 16, 16)] = z
        return carry

    lax.fori_loop(0, rows, body, 0)


DNB = 8
DROUNDS = NCHUNK // DNB
DLEFT = NCHUNK - DROUNDS * DNB


def _deg_body(dst_hbm, part_hbm, *scr):
    dst_all = scr[0]
    dst_loc = scr[1:1 + DNB]
    ones_v = scr[1 + DNB]
    zbuf = scr[2 + DNB]
    acc = scr[3 + DNB]
    sem = scr[4 + DNB:4 + 2 * DNB]
    cid = lax.axis_index("c")
    sid = lax.axis_index("s")
    base = sid * EPT
    lo = cid * HALF
    dump = HALF + sid * 8

    pltpu.sync_copy(dst_hbm.at[pl.ds(base, EPT)], dst_all)

    one = jnp.full((16,), 1.0, jnp.float32)

    def fill_ones(r, carry):
        ones_v[r, pl.ds(0, 16)] = one
        return carry

    lax.fori_loop(0, CB, fill_ones, 0)
    _zero_fill(zbuf, DRPT, 16)
    pltpu.sync_copy(zbuf, acc.at[pl.ds(sid * DRPT, DRPT)])
    plsc.subcore_barrier()

    def round_(n0, nb):
        ss = []
        for b in range(nb):
            n = n0 + b
            for k in range(CB // 16):
                v = dst_all[pl.ds(n * CB + k * 16, 16)] - lo
                ok = (v >= 0) & (v < HALF)
                dst_loc[b][pl.ds(k * 16, 16)] = jnp.where(ok, v, dump)
            ss.append(pltpu.async_copy(ones_v, acc.at[dst_loc[b]], sem[b],
                                       add=True))
        for s in ss:
            s.wait()

    def outer(h, carry):
        round_(h * DNB, DNB)
        return carry

    lax.fori_loop(0, DROUNDS, outer, 0)
    if DLEFT:
        round_(DROUNDS * DNB, DLEFT)
    plsc.subcore_barrier()
    pltpu.sync_copy(acc.at[pl.ds(sid * HRPT, HRPT)],
                    part_hbm.at[pl.ds(cid * HALF + sid * HRPT, HRPT)])


_deg_call = functools.partial(
    pl.kernel,
    out_type=jax.ShapeDtypeStruct((NP, 16), jnp.float32),
    mesh=plsc.VectorSubcoreMesh(**_MESH),
    scratch_types=(
        [pltpu.VMEM((EPT,), jnp.int32)]
        + [pltpu.VMEM((CB,), jnp.int32)] * DNB
        + [pltpu.VMEM((CB, 16), jnp.float32)]
        + [pltpu.VMEM((DRPT, 16), jnp.float32)]
        + [pltpu.VMEM_SHARED((DROWS, 16), jnp.float32)]
        + [pltpu.SemaphoreType.DMA] * DNB
    ),
)(_deg_body)


def _agg_body(g_hbm, src_hbm, dst_hbm, nr_hbm, part_hbm, *scr):
    src_all, dst_all = scr[0], scr[1]
    nr_v = scr[2]
    src_loc = scr[3:3 + NBUF]
    dst_loc = scr[3 + NBUF:3 + 2 * NBUF]
    rows = scr[3 + 2 * NBUF:3 + 3 * NBUF]
    sem = scr[3 + 3 * NBUF:3 + 4 * NBUF]
    ssem = scr[3 + 4 * NBUF:3 + 5 * NBUF]
    cid = lax.axis_index("c")
    sid = lax.axis_index("s")
    base = sid * EPT
    lo = cid * HALF
    dump = HALF + sid * 8

    def stage(n, b):
        for k in range(ACB // 16):
            src_loc[b][pl.ds(k * 16, 16)] = src_all[pl.ds(n * ACB + k * 16, 16)]
            v = dst_all[pl.ds(n * ACB + k * 16, 16)] - lo
            ok = (v >= 0) & (v < HALF)
            dst_loc[b][pl.ds(k * 16, 16)] = jnp.where(ok, v, dump)

    def fire(n, b):
        stage(n, b)
        return pltpu.async_copy(g_hbm.at[src_loc[b]], rows[b], sem[b])

    pltpu.sync_copy(src_hbm.at[pl.ds(base, EPT)], src_all)
    pltpu.sync_copy(dst_hbm.at[pl.ds(base, EPT)], dst_all)
    pltpu.sync_copy(nr_hbm.at[pl.ds(0, 16)], nr_v)

    nz = DRPT // ACB
    for b in range(NBUF):
        _zero_fill(rows[b], ACB, D)
    for j in range(nz):
        pltpu.sync_copy(rows[j % NBUF],
                        acc_of(scr).at[pl.ds(sid * DRPT + j * ACB, ACB)])
    rem = DRPT - nz * ACB
    if rem:
        pltpu.sync_copy(rows[0].at[pl.ds(0, rem)],
                        acc_of(scr).at[pl.ds(sid * DRPT + nz * ACB, rem)])

    plsc.subcore_barrier()
    acc = acc_of(scr)

    def round_(n0, nb):
        ds = [fire(n0 + b, b) for b in range(nb)]
        ss = []
        for b in range(nb):
            ds[b].wait()
            ss.append(pltpu.async_copy(rows[b], acc.at[dst_loc[b]], ssem[b],
                                       add=True))
        for s in ss:
            s.wait()

    def outer(h, carry):
        round_(h * NBUF, NBUF)
        return carry

    nr = nr_v[pl.ds(0, 16)][0]
    lax.fori_loop(0, nr, outer, 0)
    if LEFT:
        round_(ROUNDS * NBUF, LEFT)

    plsc.subcore_barrier()
    pltpu.sync_copy(acc.at[pl.ds(sid * HRPT, HRPT)],
                    part_hbm.at[pl.ds(cid * HALF + sid * HRPT, HRPT)])


def acc_of(scr):
    return scr[3 + 5 * NBUF]


_agg_call = functools.partial(
    pl.kernel,
    out_type=jax.ShapeDtypeStruct((NP, D), jnp.float32),
    mesh=plsc.VectorSubcoreMesh(**_MESH),
    scratch_types=(
        [pltpu.VMEM((EPT,), jnp.int32)] * 2
        + [pltpu.VMEM((16,), jnp.int32)]
        + [pltpu.VMEM((ACB,), jnp.int32)] * (2 * NBUF)
        + [pltpu.VMEM((ACB, D), jnp.float32)] * NBUF
        + [pltpu.SemaphoreType.DMA] * (2 * NBUF)
        + [pltpu.VMEM_SHARED((DROWS, D), jnp.float32)]
    ),
)(_agg_body)


BS = 1000
GRID = N // BS

_DOT = dict(preferred_element_type=jnp.float32,
            precision=jax.lax.Precision.HIGHEST)

_ROW = pl.BlockSpec((BS, D), lambda i: (i, 0))
_FULL = pl.BlockSpec((D, D), lambda i: (0, 0))
_BIAS = pl.BlockSpec((1, D), lambda i: (0, 0))
_OUT = jax.ShapeDtypeStruct((N, D), jnp.float32)


def _dinv_body(degp_ref, dinv_ref):
    deg = jnp.sum(degp_ref[...], axis=1) / 16.0 + 1.0
    dinv_ref[...] = jnp.broadcast_to(lax.rsqrt(deg)[:, None], (BS, D))


def _dinv(degp):
    return pl.pallas_call(
        _dinv_body,
        grid=(GRID,),
        in_specs=[pl.BlockSpec((BS, 16), lambda i: (i, 0))],
        out_specs=_ROW,
        out_shape=_OUT,
    )(degp)


def _mm_body(a_ref, w_ref, dinv_ref, g_ref):
    g_ref[...] = jnp.dot(a_ref[...], w_ref[...], **_DOT) * dinv_ref[...]


def _mm(a, w, dinvb):
    return pl.pallas_call(
        _mm_body,
        grid=(GRID,),
        in_specs=[_ROW, _FULL, _ROW],
        out_specs=_ROW,
        out_shape=_OUT,
    )(a, w, dinvb)


def _combine_body(p_ref, g_ref, dinv_ref, b_ref, a_ref):
    a_ref[...] = jnp.maximum(
        dinv_ref[...] * (p_ref[...] + g_ref[...]) + b_ref[...], 0.0)


def _combine(p, g, dinvb, b):
    return pl.pallas_call(
        _combine_body,
        grid=(GRID,),
        in_specs=[_ROW, _ROW, _ROW, _BIAS],
        out_specs=_ROW,
        out_shape=_OUT,
    )(p, g, dinvb, b)


def _layer_body(p_ref, g_ref, dinv_ref, b_ref, w_ref, gn_ref):
    dinv = dinv_ref[...]
    a = jnp.maximum(dinv * (p_ref[...] + g_ref[...]) + b_ref[...], 0.0)
    gn_ref[...] = jnp.dot(a, w_ref[...], **_DOT) * dinv


def _layer(p, g, dinvb, b, w):
    return pl.pallas_call(
        _layer_body,
        grid=(GRID,),
        in_specs=[_ROW, _ROW, _ROW, _BIAS, _FULL],
        out_specs=_ROW,
        out_shape=_OUT,
    )(p, g, dinvb, b, w)


def kernel(x, edge_index, W1, b1, W2, b2, W3, b3):
    src = edge_index[0].astype(jnp.int32)
    dst = edge_index[1].astype(jnp.int32)

    degp = _deg_call(dst)
    nrarr = jnp.full((16,), ROUNDS, jnp.int32)
    dinvb = _dinv(degp)

    g = _mm(x, W1, dinvb)
    p = _agg_call(g, src, dst, nrarr)
    for (b, w) in ((b1, W2), (b2, W3)):
        g = _layer(p[:N], g, dinvb, b.reshape(1, D), w)
        p = _agg_call(g, src, dst, nrarr)
    return _combine(p[:N], g, dinvb, b3.reshape(1, D))
